# Initial kernel scaffold; baseline (speedup 1.0000x reference)
import jax
import jax.numpy as jnp
from jax.experimental import pallas as pl


def kernel(input_ids, embedding_weight):
    raise NotImplementedError("write your pallas kernel here")


# speedup vs baseline: 1.3061x; 1.3061x over previous
import functools

import jax
import jax.numpy as jnp
from jax import lax
from jax.experimental import pallas as pl
from jax.experimental.pallas import tpu as pltpu
from jax.experimental.pallas import tpu_sc as plsc

CH = 128


def kernel(input_ids, embedding_weight):
    B, L = input_ids.shape
    V, D = embedding_weight.shape
    N = B * L

    info = plsc.get_sparse_core_info()
    NC, NS = info.num_cores, info.num_subcores
    NW = NC * NS
    assert N % (NW * CH) == 0
    n_per_w = N // NW
    nch = n_per_w // CH

    ids3 = input_ids.reshape(NW, nch, CH).astype(jnp.int32)
    mesh = plsc.VectorSubcoreMesh(core_axis_name="c", subcore_axis_name="s")

    @functools.partial(
        pl.kernel,
        mesh=mesh,
        out_type=jax.ShapeDtypeStruct((N, D), jnp.float32),
        scratch_types=[
            pltpu.VMEM((nch, CH), jnp.int32),
            pltpu.VMEM((CH, D), jnp.float32),
            pltpu.SemaphoreType.DMA,
        ],
        compiler_params=pltpu.CompilerParams(use_tc_tiling_on_sc=False),
    )
    def emb(ids_hbm, table_hbm, out_hbm, idx_v, buf, gsem):
        wid = lax.axis_index("s") * NC + lax.axis_index("c")
        base = wid * n_per_w
        pltpu.sync_copy(ids_hbm.at[wid], idx_v)

        def body(c, carry):
            pltpu.async_copy(table_hbm.at[idx_v.at[c]], buf, gsem).wait()
            pltpu.sync_copy(buf, out_hbm.at[pl.ds(base + c * CH, CH)])
            return carry

        lax.fori_loop(0, nch, body, 0)

    out = emb(ids3, embedding_weight)
    return out.reshape(B, L, D)


# speedup vs baseline: 1.4757x; 1.1298x over previous
import functools

import jax
import jax.numpy as jnp
from jax import lax
from jax.experimental import pallas as pl
from jax.experimental.pallas import tpu as pltpu
from jax.experimental.pallas import tpu_sc as plsc

GC = 1024


def kernel(input_ids, embedding_weight):
    B, L = input_ids.shape
    V, D = embedding_weight.shape
    N = B * L

    info = plsc.get_sparse_core_info()
    NC, NS = info.num_cores, info.num_subcores
    NW = NC * NS
    assert N % (NW * GC) == 0
    n_per_w = N // NW

    ids3 = input_ids.reshape(NW, n_per_w).astype(jnp.int32)
    mesh = plsc.VectorSubcoreMesh(core_axis_name="c", subcore_axis_name="s")

    @functools.partial(
        pl.kernel,
        mesh=mesh,
        out_type=jax.ShapeDtypeStruct((N, D), jnp.float32),
        scratch_types=[
            pltpu.VMEM((n_per_w,), jnp.int32),
            pltpu.VMEM((GC, D), jnp.float32),
            pltpu.SemaphoreType.DMA,
        ],
        compiler_params=pltpu.CompilerParams(use_tc_tiling_on_sc=False),
    )
    def emb(ids_hbm, table_hbm, out_hbm, idx_v, buf, gsem):
        wid = lax.axis_index("s") * NC + lax.axis_index("c")
        base = wid * n_per_w
        pltpu.sync_copy(ids_hbm.at[wid], idx_v)

        def body(g, carry):
            pltpu.async_copy(
                table_hbm.at[idx_v.at[pl.ds(g * GC, GC)]], buf, gsem
            ).wait()
            pltpu.sync_copy(buf, out_hbm.at[pl.ds(base + g * GC, GC)])
            return carry

        lax.fori_loop(0, n_per_w // GC, body, 0)

    out = emb(ids3, embedding_weight)
    return out.reshape(B, L, D)


# speedup vs baseline: 1.5002x; 1.0166x over previous
import functools

import jax
import jax.numpy as jnp
from jax import lax
from jax.experimental import pallas as pl
from jax.experimental.pallas import tpu as pltpu
from jax.experimental.pallas import tpu_sc as plsc

GC = 640
K = 4


def kernel(input_ids, embedding_weight):
    B, L = input_ids.shape
    V, D = embedding_weight.shape
    N = B * L

    info = plsc.get_sparse_core_info()
    NC, NS = info.num_cores, info.num_subcores
    NW = NC * NS
    assert N % (NW * GC * K) == 0
    n_per_w = N // NW
    ngrp = n_per_w // GC
    nblk = ngrp // K

    ids3 = input_ids.reshape(NW, n_per_w).astype(jnp.int32)
    mesh = plsc.VectorSubcoreMesh(core_axis_name="c", subcore_axis_name="s")

    @functools.partial(
        pl.kernel,
        mesh=mesh,
        out_type=jax.ShapeDtypeStruct((N, D), jnp.float32),
        scratch_types=[
            pltpu.VMEM((n_per_w,), jnp.int32),
            *[pltpu.VMEM((GC, D), jnp.float32) for _ in range(K)],
            *[pltpu.SemaphoreType.DMA for _ in range(K)],
        ],
        compiler_params=pltpu.CompilerParams(use_tc_tiling_on_sc=False),
    )
    def emb(ids_hbm, table_hbm, out_hbm, idx_v, *bufsem):
        bufs, sems = bufsem[:K], bufsem[K:]
        wid = lax.axis_index("s") * NC + lax.axis_index("c")
        base = wid * n_per_w
        pltpu.sync_copy(ids_hbm.at[wid], idx_v)

        def gather(c, k):
            pltpu.async_copy(
                table_hbm.at[idx_v.at[pl.ds(c * GC, GC)]], bufs[k], sems[k]
            )

        def gwait(k):
            pltpu.make_async_copy(
                table_hbm.at[idx_v.at[pl.ds(0, GC)]], bufs[k], sems[k]
            ).wait()

        def store(c, k):
            pltpu.sync_copy(bufs[k], out_hbm.at[pl.ds(base + c * GC, GC)])

        for k in range(K):
            gather(k, k)

        def body(blk, carry):
            for k in range(K):
                c = blk * K + k
                gwait(k)
                store(c, k)
                gather(c + K, k)
            return carry

        lax.fori_loop(0, nblk - 1, body, 0)
        for k in range(K):
            c = (nblk - 1) * K + k
            gwait(k)
            store(c, k)

    out = emb(ids3, embedding_weight)
    return out.reshape(B, L, D)


# speedup vs baseline: 2.0486x; 1.3655x over previous
import functools

import jax
import jax.numpy as jnp
from jax import lax
from jax.experimental import pallas as pl
from jax.experimental.pallas import tpu as pltpu
from jax.experimental.pallas import tpu_sc as plsc

GC = 640
K = 4


def kernel(input_ids, embedding_weight):
    B, L = input_ids.shape
    V, D = embedding_weight.shape
    N = B * L

    info = plsc.get_sparse_core_info()
    NC, NS = info.num_cores, info.num_subcores
    NW = NC * NS
    assert N % (NW * GC * K) == 0
    n_per_w = N // NW
    ngrp = n_per_w // GC
    nblk = ngrp // K

    ids3 = input_ids.reshape(NW, n_per_w).astype(jnp.int32)
    mesh = plsc.VectorSubcoreMesh(core_axis_name="c", subcore_axis_name="s")

    @functools.partial(
        pl.kernel,
        mesh=mesh,
        out_type=jax.ShapeDtypeStruct((N, 128), jnp.float32),
        scratch_types=[
            pltpu.VMEM((n_per_w,), jnp.int32),
            *[pltpu.VMEM((GC, D), jnp.float32) for _ in range(K)],
            *[pltpu.SemaphoreType.DMA for _ in range(K)],
        ],
        compiler_params=pltpu.CompilerParams(use_tc_tiling_on_sc=False),
    )
    def emb(ids_hbm, table_hbm, out_hbm, idx_v, *bufsem):
        bufs, sems = bufsem[:K], bufsem[K:]
        wid = lax.axis_index("s") * NC + lax.axis_index("c")
        base = wid * n_per_w
        pltpu.sync_copy(ids_hbm.at[wid], idx_v)

        def gather(c, k):
            pltpu.async_copy(
                table_hbm.at[idx_v.at[pl.ds(c * GC, GC)]], bufs[k], sems[k]
            )

        def gwait(k):
            pltpu.make_async_copy(
                table_hbm.at[idx_v.at[pl.ds(0, GC)]], bufs[k], sems[k]
            ).wait()

        def store(c, k):
            pltpu.sync_copy(
                bufs[k], out_hbm.at[pl.ds(base + c * GC, GC), pl.ds(0, D)]
            )

        for k in range(K):
            gather(k, k)

        def body(blk, carry):
            for k in range(K):
                c = blk * K + k
                gwait(k)
                store(c, k)
                gather(c + K, k)
            return carry

        lax.fori_loop(0, nblk - 1, body, 0)
        for k in range(K):
            c = (nblk - 1) * K + k
            gwait(k)
            store(c, k)

    out = emb(ids3, embedding_weight)
    return out[:, :D].reshape(B, L, D)
